# baseline (speedup 1.0000x reference)
import functools

import jax
import jax.numpy as jnp
from jax import lax
from jax.experimental import pallas as pl
from jax.experimental.pallas import tpu as pltpu
from jax.experimental.pallas import tpu_sc as plsc


def _tc_body(q_ref, es_ref, idx_ref, loss_ref, *, n_codes, scale):
    qt = q_ref[...].T
    es = es_ref[...]
    e2 = 0.25 * jnp.sum(es * es, axis=1, keepdims=True)
    q2 = jnp.sum(qt * qt, axis=0, keepdims=True)
    mm = lax.dot_general(
        es, qt, (((1,), (0,)), ((), ())),
        preferred_element_type=jnp.float32)
    scores = (q2 + e2) + mm
    m = jnp.min(scores, axis=0)
    code_iota = lax.broadcasted_iota(
        jnp.int32, scores.shape, 0).astype(jnp.float32)
    idx = jnp.min(
        jnp.where(scores == m[None, :], code_iota, float(n_codes)),
        axis=0).astype(jnp.int32)
    idx_ref[0, 0, :] = idx

    partial = (jnp.sum(m) * scale).reshape(1, 1)

    @pl.when(pl.program_id(0) == 0)
    def _init():
        loss_ref[...] = jnp.zeros_like(loss_ref)

    loss_ref[...] += partial


def _tc_stage(q2d, embedding, tile, n_total):
    n, c = q2d.shape
    k = embedding.shape[0]
    es = embedding * (-2.0)
    grid = n // tile
    scale = 1.25 / float(n_total * c)
    idx3, loss = pl.pallas_call(
        functools.partial(_tc_body, n_codes=k, scale=scale),
        grid=(grid,),
        in_specs=[
            pl.BlockSpec((tile, c), lambda i: (i, 0)),
            pl.BlockSpec((k, c), lambda i: (0, 0)),
        ],
        out_specs=[
            pl.BlockSpec((1, 1, tile), lambda i: (i, 0, 0)),
            pl.BlockSpec((1, 1), lambda i: (0, 0)),
        ],
        out_shape=[
            jax.ShapeDtypeStruct((grid, 1, tile), jnp.int32),
            jax.ShapeDtypeStruct((1, 1), jnp.float32),
        ],
    )(q2d, es)
    return idx3.reshape(n), loss[0, 0]


_CHUNK = 128


def _sc_gather(table128, idx, n):
    info = plsc.get_sparse_core_info()
    nw = info.num_cores * info.num_subcores
    b_per_w = n // nw
    n_ch = b_per_w // _CHUNK
    idx3 = idx.reshape(nw, n_ch, _CHUNK)
    mesh = plsc.VectorSubcoreMesh(core_axis_name="c", subcore_axis_name="s")

    @functools.partial(
        pl.kernel, mesh=mesh,
        out_type=jax.ShapeDtypeStruct((n, 128), jnp.float32),
        scratch_types=[
            pltpu.VMEM((n_ch, _CHUNK), jnp.int32),
            pltpu.VMEM((b_per_w // 2, 128), jnp.float32),
            pltpu.SemaphoreType.DMA,
        ],
    )
    def gather(table_hbm, idx_hbm, out_hbm, idx_v, rows_v, sem):
        wid = lax.axis_index("s") * info.num_cores + lax.axis_index("c")
        pltpu.sync_copy(idx_hbm.at[wid], idx_v)
        half = n_ch // 2
        for r in range(2):
            copies = []
            for j in range(half):
                copies.append(pltpu.async_copy(
                    table_hbm.at[idx_v.at[r * half + j]],
                    rows_v.at[pl.ds(j * _CHUNK, _CHUNK)],
                    sem))
            for cp in copies:
                cp.wait()
            pltpu.sync_copy(
                rows_v,
                out_hbm.at[pl.ds(wid * b_per_w + r * (b_per_w // 2),
                                 b_per_w // 2)])

    return gather(table128, idx3)


def kernel(queries, embedding):
    b, t, c = queries.shape
    n = b * t
    q2d = queries.reshape(n, c)
    idx, vq_loss = _tc_stage(q2d, embedding, tile=2048, n_total=n)
    table128 = jnp.pad(embedding, ((0, 0), (0, 128 - c)))
    quantized = _sc_gather(table128, idx, n)[:, :c]
    return idx.reshape(b, t), quantized.reshape(b, t, c), vq_loss


# speedup vs baseline: 1.0323x; 1.0323x over previous
import functools

import jax
import jax.numpy as jnp
from jax import lax
from jax.experimental import pallas as pl
from jax.experimental.pallas import tpu as pltpu
from jax.experimental.pallas import tpu_sc as plsc


def _tc_body(qt_ref, es_ref, idx_ref, loss_ref, *, n_codes, scale):
    qt = qt_ref[...]
    es = es_ref[...]
    e2 = 0.25 * jnp.sum(es * es, axis=1, keepdims=True)
    q2 = jnp.sum(qt * qt, axis=0, keepdims=True)
    mm = lax.dot_general(
        es, qt, (((1,), (0,)), ((), ())),
        preferred_element_type=jnp.float32)
    scores = (q2 + e2) + mm
    m = jnp.min(scores, axis=0)
    code_iota = lax.broadcasted_iota(
        jnp.int32, scores.shape, 0).astype(jnp.float32)
    idx = jnp.min(
        jnp.where(scores == m[None, :], code_iota, float(n_codes)),
        axis=0).astype(jnp.int32)
    idx_ref[0, 0, :] = idx

    partial = (jnp.sum(m) * scale).reshape(1, 1)

    @pl.when(pl.program_id(0) == 0)
    def _init():
        loss_ref[...] = jnp.zeros_like(loss_ref)

    loss_ref[...] += partial


def _tc_stage(q2d, embedding, tile, n_total):
    n, c = q2d.shape
    k = embedding.shape[0]
    qt = q2d.T
    es = embedding * (-2.0)
    grid = n // tile
    scale = 1.25 / float(n_total * c)
    idx3, loss = pl.pallas_call(
        functools.partial(_tc_body, n_codes=k, scale=scale),
        grid=(grid,),
        in_specs=[
            pl.BlockSpec((c, tile), lambda i: (0, i)),
            pl.BlockSpec((k, c), lambda i: (0, 0)),
        ],
        out_specs=[
            pl.BlockSpec((1, 1, tile), lambda i: (i, 0, 0)),
            pl.BlockSpec((1, 1), lambda i: (0, 0)),
        ],
        out_shape=[
            jax.ShapeDtypeStruct((grid, 1, tile), jnp.int32),
            jax.ShapeDtypeStruct((1, 1), jnp.float32),
        ],
    )(qt, es)
    return idx3.reshape(n), loss[0, 0]


_CHUNK = 128


def _sc_gather(table128, idx, n):
    info = plsc.get_sparse_core_info()
    nw = info.num_cores * info.num_subcores
    b_per_w = n // nw
    n_ch = b_per_w // _CHUNK
    idx3 = idx.reshape(nw, n_ch, _CHUNK)
    mesh = plsc.VectorSubcoreMesh(core_axis_name="c", subcore_axis_name="s")

    @functools.partial(
        pl.kernel, mesh=mesh,
        out_type=jax.ShapeDtypeStruct((n, 128), jnp.float32),
        scratch_types=[
            pltpu.VMEM((n_ch, _CHUNK), jnp.int32),
            pltpu.VMEM((b_per_w // 2, 128), jnp.float32),
            pltpu.SemaphoreType.DMA,
        ],
    )
    def gather(table_hbm, idx_hbm, out_hbm, idx_v, rows_v, sem):
        wid = lax.axis_index("s") * info.num_cores + lax.axis_index("c")
        pltpu.sync_copy(idx_hbm.at[wid], idx_v)
        half = n_ch // 2
        for r in range(2):
            copies = []
            for j in range(half):
                copies.append(pltpu.async_copy(
                    table_hbm.at[idx_v.at[r * half + j]],
                    rows_v.at[pl.ds(j * _CHUNK, _CHUNK)],
                    sem))
            for cp in copies:
                cp.wait()
            pltpu.sync_copy(
                rows_v,
                out_hbm.at[pl.ds(wid * b_per_w + r * (b_per_w // 2),
                                 b_per_w // 2)])

    return gather(table128, idx3)


def kernel(queries, embedding):
    b, t, c = queries.shape
    n = b * t
    q2d = queries.reshape(n, c)
    idx, vq_loss = _tc_stage(q2d, embedding, tile=2048, n_total=n)
    table128 = jnp.pad(embedding, ((0, 0), (0, 128 - c)))
    quantized = _sc_gather(table128, idx, n)[:, :c]
    return idx.reshape(b, t), quantized.reshape(b, t, c), vq_loss


# speedup vs baseline: 1.0550x; 1.0220x over previous
import functools

import jax
import jax.numpy as jnp
from jax import lax
from jax.experimental import pallas as pl
from jax.experimental.pallas import tpu as pltpu
from jax.experimental.pallas import tpu_sc as plsc


def _tc_body(qt_ref, es_ref, idx_ref, loss_ref, *, n_codes, scale):
    qt = qt_ref[...]
    es = es_ref[...]
    e2 = 0.25 * jnp.sum(es * es, axis=1, keepdims=True)
    q2 = jnp.sum(qt * qt, axis=0, keepdims=True)
    mm = lax.dot_general(
        es, qt, (((1,), (0,)), ((), ())),
        preferred_element_type=jnp.float32)
    scores = (q2 + e2) + mm
    m = jnp.min(scores, axis=0)
    code_iota = lax.broadcasted_iota(
        jnp.int32, scores.shape, 0).astype(jnp.float32)
    idx = jnp.min(
        jnp.where(scores == m[None, :], code_iota, float(n_codes)),
        axis=0).astype(jnp.int32)
    idx_ref[0, 0, :] = idx

    partial = (jnp.sum(m) * scale).reshape(1, 1)

    @pl.when(pl.program_id(0) == 0)
    def _init():
        loss_ref[...] = jnp.zeros_like(loss_ref)

    loss_ref[...] += partial


def _tc_stage(q2d, embedding, tile, n_total):
    n, c = q2d.shape
    k = embedding.shape[0]
    qt = q2d.T
    es = embedding * (-2.0)
    grid = n // tile
    scale = 1.25 / float(n_total * c)
    idx3, loss = pl.pallas_call(
        functools.partial(_tc_body, n_codes=k, scale=scale),
        grid=(grid,),
        in_specs=[
            pl.BlockSpec((c, tile), lambda i: (0, i)),
            pl.BlockSpec((k, c), lambda i: (0, 0)),
        ],
        out_specs=[
            pl.BlockSpec((1, 1, tile), lambda i: (i, 0, 0)),
            pl.BlockSpec((1, 1), lambda i: (0, 0)),
        ],
        out_shape=[
            jax.ShapeDtypeStruct((grid, 1, tile), jnp.int32),
            jax.ShapeDtypeStruct((1, 1), jnp.float32),
        ],
    )(qt, es)
    return idx3.reshape(n), loss[0, 0]


_CHUNK = 128


def _sc_gather(table128, idx, n):
    info = plsc.get_sparse_core_info()
    nw = info.num_cores * info.num_subcores
    b_per_w = n // nw
    n_ch = b_per_w // _CHUNK
    idx3 = idx.reshape(nw, n_ch, _CHUNK)
    mesh = plsc.VectorSubcoreMesh(core_axis_name="c", subcore_axis_name="s")

    @functools.partial(
        pl.kernel, mesh=mesh,
        out_type=jax.ShapeDtypeStruct((n, 128), jnp.float32),
        scratch_types=[
            pltpu.VMEM((n_ch, _CHUNK), jnp.int32),
            pltpu.VMEM((b_per_w // 2, 128), jnp.float32),
            pltpu.SemaphoreType.DMA,
        ],
    )
    def gather(table_hbm, idx_hbm, out_hbm, idx_v, rows_v, sem):
        wid = lax.axis_index("s") * info.num_cores + lax.axis_index("c")
        pltpu.sync_copy(idx_hbm.at[wid], idx_v)
        half = n_ch // 2
        for r in range(2):
            copies = []
            for j in range(half):
                copies.append(pltpu.async_copy(
                    table_hbm.at[idx_v.at[r * half + j]],
                    rows_v.at[pl.ds(j * _CHUNK, _CHUNK)],
                    sem))
            for cp in copies:
                cp.wait()
            pltpu.sync_copy(
                rows_v,
                out_hbm.at[pl.ds(wid * b_per_w + r * (b_per_w // 2),
                                 b_per_w // 2)])

    return gather(table128, idx3)


def kernel(queries, embedding):
    b, t, c = queries.shape
    n = b * t
    q2d = queries.reshape(n, c)
    idx, vq_loss = _tc_stage(q2d, embedding, tile=4096, n_total=n)
    table128 = jnp.pad(embedding, ((0, 0), (0, 128 - c)))
    quantized = _sc_gather(table128, idx, n)[:, :c]
    return idx.reshape(b, t), quantized.reshape(b, t, c), vq_loss


# speedup vs baseline: 1.0639x; 1.0084x over previous
import functools

import jax
import jax.numpy as jnp
from jax import lax
from jax.experimental import pallas as pl
from jax.experimental.pallas import tpu as pltpu
from jax.experimental.pallas import tpu_sc as plsc


def _tc_body(qt_ref, es_ref, idx_ref, loss_ref, *, n_codes, scale):
    qt = qt_ref[...]
    es = es_ref[...]
    e2 = 0.25 * jnp.sum(es * es, axis=1, keepdims=True)
    q2 = jnp.sum(qt * qt, axis=0, keepdims=True)
    mm = lax.dot_general(
        es, qt, (((1,), (0,)), ((), ())),
        preferred_element_type=jnp.float32)
    scores = (q2 + e2) + mm
    m = jnp.min(scores, axis=0)
    code_iota = lax.broadcasted_iota(
        jnp.int32, scores.shape, 0).astype(jnp.float32)
    idx = jnp.min(
        jnp.where(scores == m[None, :], code_iota, float(n_codes)),
        axis=0).astype(jnp.int32)
    idx_ref[0, 0, :] = idx

    partial = (jnp.sum(m) * scale).reshape(1, 1)

    @pl.when(pl.program_id(0) == 0)
    def _init():
        loss_ref[...] = jnp.zeros_like(loss_ref)

    loss_ref[...] += partial


def _tc_stage(q2d, embedding, tile, n_total):
    n, c = q2d.shape
    k = embedding.shape[0]
    qt = q2d.T
    es = embedding * (-2.0)
    grid = n // tile
    scale = 1.25 / float(n_total * c)
    idx3, loss = pl.pallas_call(
        functools.partial(_tc_body, n_codes=k, scale=scale),
        grid=(grid,),
        in_specs=[
            pl.BlockSpec((c, tile), lambda i: (0, i)),
            pl.BlockSpec((k, c), lambda i: (0, 0)),
        ],
        out_specs=[
            pl.BlockSpec((1, 1, tile), lambda i: (i, 0, 0)),
            pl.BlockSpec((1, 1), lambda i: (0, 0)),
        ],
        out_shape=[
            jax.ShapeDtypeStruct((grid, 1, tile), jnp.int32),
            jax.ShapeDtypeStruct((1, 1), jnp.float32),
        ],
    )(qt, es)
    return idx3.reshape(n), loss[0, 0]


_CHUNK = 128


def _sc_gather(table128, idx, n):
    info = plsc.get_sparse_core_info()
    nw = info.num_cores * info.num_subcores
    b_per_w = n // nw
    n_ch = b_per_w // _CHUNK
    idx3 = idx.reshape(nw, n_ch, _CHUNK)
    mesh = plsc.VectorSubcoreMesh(core_axis_name="c", subcore_axis_name="s")

    @functools.partial(
        pl.kernel, mesh=mesh,
        out_type=jax.ShapeDtypeStruct((n, 128), jnp.float32),
        scratch_types=[
            pltpu.VMEM((n_ch, _CHUNK), jnp.int32),
            pltpu.VMEM((b_per_w // 2, 128), jnp.float32),
            pltpu.SemaphoreType.DMA,
        ],
    )
    def gather(table_hbm, idx_hbm, out_hbm, idx_v, rows_v, sem):
        wid = lax.axis_index("s") * info.num_cores + lax.axis_index("c")
        pltpu.sync_copy(idx_hbm.at[wid], idx_v)
        half = n_ch // 2
        for r in range(2):
            copies = []
            for j in range(half):
                copies.append(pltpu.async_copy(
                    table_hbm.at[idx_v.at[r * half + j]],
                    rows_v.at[pl.ds(j * _CHUNK, _CHUNK)],
                    sem))
            for cp in copies:
                cp.wait()
            pltpu.sync_copy(
                rows_v,
                out_hbm.at[pl.ds(wid * b_per_w + r * (b_per_w // 2),
                                 b_per_w // 2)])

    return gather(table128, idx3)


def kernel(queries, embedding):
    b, t, c = queries.shape
    n = b * t
    q2d = queries.reshape(n, c)
    idx, vq_loss = _tc_stage(q2d, embedding, tile=8192, n_total=n)
    table128 = jnp.pad(embedding, ((0, 0), (0, 128 - c)))
    quantized = _sc_gather(table128, idx, n)[:, :c]
    return idx.reshape(b, t), quantized.reshape(b, t, c), vq_loss
